# baseline (speedup 1.0000x reference)
import jax
import jax.numpy as jnp
from jax import lax
from jax.experimental import pallas as pl
from jax.experimental.pallas import tpu as pltpu
from jax.experimental.pallas import tpu_sc as plsc

_N_USER = 60000
_N_ITEM = 40000
_N_NODES = _N_USER + _N_ITEM
_N_EDGES = 1600000
_HALF = 16
_N_LAYERS = 3
_BATCH = 4096

_CHUNK = 128
_SUB = 2
_SUB_E = _SUB * _CHUNK
_NS = 16
_NSB = _N_EDGES // _SUB_E
_NV = 392
_NB = 3 * _BATCH
_NB_CHUNKS = _NB // _CHUNK
_N_PAD = 100096
_ROWS_PER_SUB = _N_PAD // _NS
_ZCHUNK = 272
_NBUF = 4


def _sc_body(ego0_hbm, src_hbm, dst_hbm, val_hbm, bidx_hbm,
             egos_hbm, outb_hbm, acc,
             s0, s1, s2, s3, d0, d1, d2, d3, v0, v1, v2, v3,
             r0, r1, r2, r3, zbuf,
             i0, i1, i2, i3, g0, g1, g2, g3, c0, c1, c2, c3):
    src_b = (s0, s1, s2, s3)
    dst_b = (d0, d1, d2, d3)
    val_b = (v0, v1, v2, v3)
    rows_b = (r0, r1, r2, r3)
    sem_i = (i0, i1, i2, i3)
    sem_g = (g0, g1, g2, g3)
    sem_s = (c0, c1, c2, c3)

    cid = lax.axis_index("c")
    sid = lax.axis_index("s")
    base_rows = sid * _ROWS_PER_SUB
    dummy = egos_hbm.at[0, 0, pl.ds(0, _SUB_E)]

    def blk(v):
        return sid + _NS * v

    def valid(v):
        return blk(v) < _NSB

    def issue_idx(v, S):
        b = blk(v)
        pltpu.async_copy(src_hbm.at[pl.ds(b * _SUB_E, _SUB_E)],
                         src_b[S], sem_i[S])
        pltpu.async_copy(dst_hbm.at[b], dst_b[S], sem_i[S])
        pltpu.async_copy(val_hbm.at[pl.ds(b * _SUB_E, _SUB_E)],
                         val_b[S], sem_i[S])

    def drain_idx(S):
        pltpu.make_async_copy(
            src_hbm.at[pl.ds(0, _SUB_E)], src_b[S], sem_i[S]).wait()
        pltpu.make_async_copy(dst_hbm.at[0], dst_b[S], sem_i[S]).wait()
        pltpu.make_async_copy(
            val_hbm.at[pl.ds(0, _SUB_E)], val_b[S], sem_i[S]).wait()

    def interleave_idx(S):
        @pl.loop(0, _SUB_E, step=_HALF)
        def _(g):
            sl = pl.ds(g, _HALF)
            src_b[S].at[sl][...] = src_b[S].at[sl][...] * 2 + cid

    def issue_gathers(tbl, S):
        @pl.loop(0, _SUB)
        def _(k):
            pltpu.async_copy(
                tbl.at[src_b[S].at[pl.ds(k * _CHUNK, _CHUNK)]],
                rows_b[S].at[pl.ds(k * _CHUNK, _CHUNK)], sem_g[S])

    def drain_gathers(S):
        pltpu.make_async_copy(dummy, rows_b[S], sem_g[S]).wait()

    def multiply(S):
        rows, val = rows_b[S], val_b[S]

        @plsc.parallel_loop(0, _SUB_E, step=_HALF, unroll=4)
        def _(c):
            vv = val.at[pl.ds(c, _HALF)][...]
            for jj in range(_HALF):
                rows.at[c + jj][...] = rows.at[c + jj][...] * vv[jj]

    def issue_scatters(S):
        @pl.loop(0, _SUB)
        def _(k):
            pltpu.async_copy(
                rows_b[S].at[pl.ds(k * _CHUNK, _CHUNK)],
                acc.at[dst_b[S].at[k]], sem_s[S], add=True)

    def drain_scatters(S):
        pltpu.make_async_copy(dummy, rows_b[S], sem_s[S]).wait()

    def process(tbl, v, S, interleave):
        Sp = (S - 1) % _NBUF
        S2 = (S + 2) % _NBUF

        @pl.when(valid(v))
        def _():
            drain_idx(S)
            if interleave:
                interleave_idx(S)
            issue_gathers(tbl, S)

        @pl.when(jnp.logical_and(v >= 1, valid(v - 1)))
        def _():
            drain_gathers(Sp)
            multiply(Sp)

        @pl.when(jnp.logical_and(v >= 2, valid(v - 2)))
        def _():
            drain_scatters(S2)

        @pl.when(jnp.logical_and(v + 2 < _NV, valid(v + 2)))
        def _():
            issue_idx(v + 2, S2)

        @pl.when(jnp.logical_and(v >= 1, valid(v - 1)))
        def _():
            issue_scatters(Sp)

    def run_layer(tbl, out_view, interleave):
        issue_idx(0, 0)
        issue_idx(1, 1)

        @pl.loop(0, _NV, step=_NBUF)
        def _(v0_):
            for d in range(_NBUF):
                process(tbl, v0_ + d, d, interleave)

        Sl = (_NV - 1) % _NBUF

        @pl.when(valid(_NV - 1))
        def _():
            drain_gathers(Sl)
            multiply(Sl)

        @pl.when(valid(_NV - 2))
        def _():
            drain_scatters((_NV + 2) % _NBUF)

        @pl.when(valid(_NV - 1))
        def _():
            issue_scatters(Sl)
            drain_scatters(Sl)

        plsc.subcore_barrier()

        @pl.loop(0, _ROWS_PER_SUB // _ZCHUNK)
        def _(k):
            sl = pl.ds(base_rows + k * _ZCHUNK, _ZCHUNK)
            pltpu.sync_copy(acc.at[sl], out_view.at[sl])
            pltpu.sync_copy(zbuf, acc.at[sl])

        plsc.subcore_barrier()

    @pl.loop(0, _ZCHUNK)
    def _(i):
        zbuf.at[i][...] = jnp.zeros((_HALF,), jnp.float32)

    @pl.loop(0, _ROWS_PER_SUB // _ZCHUNK)
    def _(k):
        pltpu.sync_copy(zbuf, acc.at[pl.ds(base_rows + k * _ZCHUNK, _ZCHUNK)])

    plsc.subcore_barrier()

    run_layer(ego0_hbm, egos_hbm.at[0, cid], True)

    @pl.loop(1, _N_LAYERS)
    def _(t):
        run_layer(egos_hbm.at[t - 1, cid], egos_hbm.at[t, cid], False)

    nb_per_sub = _NB_CHUNKS // _NS

    @pl.loop(sid * nb_per_sub, (sid + 1) * nb_per_sub)
    def _(j):
        bslice = pl.ds(j * _CHUNK, _CHUNK)
        csl = pl.ds(0, _CHUNK)
        pltpu.sync_copy(bidx_hbm.at[bslice], s0.at[csl])

        @pl.loop(0, _CHUNK, step=_HALF)
        def _(g):
            sl = pl.ds(g, _HALF)
            s1.at[sl][...] = s0.at[sl][...] * 2 + cid

        pltpu.async_copy(ego0_hbm.at[s1.at[csl]], r0.at[csl], g0).wait()

        for t in range(_N_LAYERS):
            pltpu.async_copy(egos_hbm.at[t, cid].at[s0.at[csl]],
                             r1.at[csl], g0).wait()

            @pl.loop(0, _CHUNK)
            def _(i):
                r0.at[i][...] = r0.at[i][...] + r1.at[i][...]

        @pl.loop(0, _CHUNK)
        def _(i):
            r0.at[i][...] = r0.at[i][...] * (1.0 / (_N_LAYERS + 1))

        pltpu.sync_copy(r0.at[csl], outb_hbm.at[cid].at[bslice])


@jax.jit
def kernel(users, pos_items, neg_items, edge_index, edge_vals,
           user_emb, item_emb):
    ego0 = jnp.concatenate([user_emb, item_emb], axis=0)
    ego0 = ego0.reshape(2 * _N_NODES, _HALF)
    src1d = edge_index[0]
    dst3d = edge_index[1].reshape(_NSB, _SUB, _CHUNK)
    val1d = edge_vals
    bidx = jnp.concatenate([users, pos_items + _N_USER, neg_items + _N_USER])

    mesh = plsc.VectorSubcoreMesh(core_axis_name="c", subcore_axis_name="s")
    cp = pltpu.CompilerParams(
        needs_layout_passes=False, use_tc_tiling_on_sc=False)
    run = pl.kernel(
        _sc_body,
        out_type=(
            jax.ShapeDtypeStruct((_N_LAYERS, 2, _N_PAD, _HALF), jnp.float32),
            jax.ShapeDtypeStruct((2, _NB, _HALF), jnp.float32),
        ),
        mesh=mesh,
        scratch_types=(
            [pltpu.VMEM_SHARED((_N_PAD, _HALF), jnp.float32)]
            + [pltpu.VMEM((_SUB_E,), jnp.int32) for _ in range(_NBUF)]
            + [pltpu.VMEM((_SUB, _CHUNK), jnp.int32) for _ in range(_NBUF)]
            + [pltpu.VMEM((_SUB_E,), jnp.float32) for _ in range(_NBUF)]
            + [pltpu.VMEM((_SUB_E, _HALF), jnp.float32) for _ in range(_NBUF)]
            + [pltpu.VMEM((_ZCHUNK, _HALF), jnp.float32)]
            + [pltpu.SemaphoreType.DMA for _ in range(3 * _NBUF)]
        ),
        compiler_params=cp,
        name="lightgcn_sc",
    )
    _, outb = run(ego0, src1d, dst3d, val1d, bidx)

    out = jnp.concatenate([outb[0], outb[1]], axis=1)
    u = out[:_BATCH]
    pos = out[_BATCH:2 * _BATCH]
    neg = out[2 * _BATCH:]
    return (u, pos, neg)


# speedup vs baseline: 1.0376x; 1.0376x over previous
import jax
import jax.numpy as jnp
from jax import lax
from jax.experimental import pallas as pl
from jax.experimental.pallas import tpu as pltpu
from jax.experimental.pallas import tpu_sc as plsc

_N_USER = 60000
_N_ITEM = 40000
_N_NODES = _N_USER + _N_ITEM
_N_EDGES = 1600000
_HALF = 16
_N_LAYERS = 3
_BATCH = 4096

_CHUNK = 128
_SUB = 2
_SUB_E = _SUB * _CHUNK
_NS = 16
_NSB = _N_EDGES // _SUB_E
_NV = 392
_NB = 3 * _BATCH
_NB_CHUNKS = _NB // _CHUNK
_N_PAD = 100096
_ROWS_PER_SUB = _N_PAD // _NS
_ZCHUNK = 272
_NBUF = 4


def _sc_body(ego0_hbm, src_hbm, dst_hbm, val_hbm, bidx_hbm,
             egos_hbm, outb_hbm, acc,
             s0, s1, s2, s3, d0, d1, d2, d3, v0, v1, v2, v3,
             r0, r1, r2, r3, zbuf,
             i0, i1, i2, i3, g0, g1, g2, g3, c0, c1, c2, c3):
    src_b = (s0, s1, s2, s3)
    dst_b = (d0, d1, d2, d3)
    val_b = (v0, v1, v2, v3)
    rows_b = (r0, r1, r2, r3)
    sem_i = (i0, i1, i2, i3)
    sem_g = (g0, g1, g2, g3)
    sem_s = (c0, c1, c2, c3)

    cid = lax.axis_index("c")
    sid = lax.axis_index("s")
    base_rows = sid * _ROWS_PER_SUB
    dummy = egos_hbm.at[0, 0, pl.ds(0, _SUB_E)]

    def blk(v):
        return sid + _NS * v

    def valid(v):
        return blk(v) < _NSB

    def issue_idx(v, S):
        b = blk(v)
        pltpu.async_copy(src_hbm.at[pl.ds(b * _SUB_E, _SUB_E)],
                         src_b[S], sem_i[S])
        pltpu.async_copy(dst_hbm.at[b], dst_b[S], sem_i[S])
        pltpu.async_copy(val_hbm.at[pl.ds(b * _SUB_E, _SUB_E)],
                         val_b[S], sem_i[S])

    def drain_idx(S):
        pltpu.make_async_copy(
            src_hbm.at[pl.ds(0, _SUB_E)], src_b[S], sem_i[S]).wait()
        pltpu.make_async_copy(dst_hbm.at[0], dst_b[S], sem_i[S]).wait()
        pltpu.make_async_copy(
            val_hbm.at[pl.ds(0, _SUB_E)], val_b[S], sem_i[S]).wait()

    def interleave_idx(S):
        @pl.loop(0, _SUB_E, step=_HALF)
        def _(g):
            sl = pl.ds(g, _HALF)
            src_b[S].at[sl][...] = src_b[S].at[sl][...] * 2 + cid

    def issue_gathers(tbl, S):
        @pl.loop(0, _SUB)
        def _(k):
            pltpu.async_copy(
                tbl.at[src_b[S].at[pl.ds(k * _CHUNK, _CHUNK)]],
                rows_b[S].at[pl.ds(k * _CHUNK, _CHUNK)], sem_g[S])

    def drain_gathers(S):
        pltpu.make_async_copy(dummy, rows_b[S], sem_g[S]).wait()

    def multiply(S):
        rows, val = rows_b[S], val_b[S]

        @plsc.parallel_loop(0, _SUB_E, step=_HALF, unroll=2)
        def _(c):
            vv = val.at[pl.ds(c, _HALF)][...]
            for jj in range(_HALF):
                rows.at[c + jj][...] = rows.at[c + jj][...] * vv[jj]

    def issue_scatters(S):
        @pl.loop(0, _SUB)
        def _(k):
            pltpu.async_copy(
                rows_b[S].at[pl.ds(k * _CHUNK, _CHUNK)],
                acc.at[dst_b[S].at[k]], sem_s[S], add=True)

    def drain_scatters(S):
        pltpu.make_async_copy(dummy, rows_b[S], sem_s[S]).wait()

    def process(tbl, v, S, interleave):
        Sp = (S - 1) % _NBUF
        S2 = (S + 2) % _NBUF

        @pl.when(valid(v))
        def _():
            drain_idx(S)
            if interleave:
                interleave_idx(S)
            issue_gathers(tbl, S)

        @pl.when(jnp.logical_and(v >= 1, valid(v - 1)))
        def _():
            drain_gathers(Sp)
            multiply(Sp)

        @pl.when(jnp.logical_and(v >= 2, valid(v - 2)))
        def _():
            drain_scatters(S2)

        @pl.when(jnp.logical_and(v + 2 < _NV, valid(v + 2)))
        def _():
            issue_idx(v + 2, S2)

        @pl.when(jnp.logical_and(v >= 1, valid(v - 1)))
        def _():
            issue_scatters(Sp)

    def run_layer(tbl, out_view, interleave):
        issue_idx(0, 0)
        issue_idx(1, 1)

        @pl.loop(0, _NV, step=_NBUF)
        def _(v0_):
            for d in range(_NBUF):
                process(tbl, v0_ + d, d, interleave)

        Sl = (_NV - 1) % _NBUF

        @pl.when(valid(_NV - 1))
        def _():
            drain_gathers(Sl)
            multiply(Sl)

        @pl.when(valid(_NV - 2))
        def _():
            drain_scatters((_NV + 2) % _NBUF)

        @pl.when(valid(_NV - 1))
        def _():
            issue_scatters(Sl)
            drain_scatters(Sl)

        plsc.subcore_barrier()

        @pl.loop(0, _ROWS_PER_SUB // _ZCHUNK)
        def _(k):
            sl = pl.ds(base_rows + k * _ZCHUNK, _ZCHUNK)
            pltpu.sync_copy(acc.at[sl], out_view.at[sl])
            pltpu.sync_copy(zbuf, acc.at[sl])

        plsc.subcore_barrier()

    @pl.loop(0, _ZCHUNK)
    def _(i):
        zbuf.at[i][...] = jnp.zeros((_HALF,), jnp.float32)

    @pl.loop(0, _ROWS_PER_SUB // _ZCHUNK)
    def _(k):
        pltpu.sync_copy(zbuf, acc.at[pl.ds(base_rows + k * _ZCHUNK, _ZCHUNK)])

    plsc.subcore_barrier()

    run_layer(ego0_hbm, egos_hbm.at[0, cid], True)

    @pl.loop(1, _N_LAYERS)
    def _(t):
        run_layer(egos_hbm.at[t - 1, cid], egos_hbm.at[t, cid], False)

    nb_per_sub = _NB_CHUNKS // _NS

    @pl.loop(sid * nb_per_sub, (sid + 1) * nb_per_sub)
    def _(j):
        bslice = pl.ds(j * _CHUNK, _CHUNK)
        csl = pl.ds(0, _CHUNK)
        pltpu.sync_copy(bidx_hbm.at[bslice], s0.at[csl])

        @pl.loop(0, _CHUNK, step=_HALF)
        def _(g):
            sl = pl.ds(g, _HALF)
            s1.at[sl][...] = s0.at[sl][...] * 2 + cid

        pltpu.async_copy(ego0_hbm.at[s1.at[csl]], r0.at[csl], g0).wait()

        for t in range(_N_LAYERS):
            pltpu.async_copy(egos_hbm.at[t, cid].at[s0.at[csl]],
                             r1.at[csl], g0).wait()

            @pl.loop(0, _CHUNK)
            def _(i):
                r0.at[i][...] = r0.at[i][...] + r1.at[i][...]

        @pl.loop(0, _CHUNK)
        def _(i):
            r0.at[i][...] = r0.at[i][...] * (1.0 / (_N_LAYERS + 1))

        pltpu.sync_copy(r0.at[csl], outb_hbm.at[cid].at[bslice])


@jax.jit
def kernel(users, pos_items, neg_items, edge_index, edge_vals,
           user_emb, item_emb):
    ego0 = jnp.concatenate([user_emb, item_emb], axis=0)
    ego0 = ego0.reshape(2 * _N_NODES, _HALF)
    src1d = edge_index[0]
    dst3d = edge_index[1].reshape(_NSB, _SUB, _CHUNK)
    val1d = edge_vals
    bidx = jnp.concatenate([users, pos_items + _N_USER, neg_items + _N_USER])

    mesh = plsc.VectorSubcoreMesh(core_axis_name="c", subcore_axis_name="s")
    cp = pltpu.CompilerParams(
        needs_layout_passes=False, use_tc_tiling_on_sc=False)
    run = pl.kernel(
        _sc_body,
        out_type=(
            jax.ShapeDtypeStruct((_N_LAYERS, 2, _N_PAD, _HALF), jnp.float32),
            jax.ShapeDtypeStruct((2, _NB, _HALF), jnp.float32),
        ),
        mesh=mesh,
        scratch_types=(
            [pltpu.VMEM_SHARED((_N_PAD, _HALF), jnp.float32)]
            + [pltpu.VMEM((_SUB_E,), jnp.int32) for _ in range(_NBUF)]
            + [pltpu.VMEM((_SUB, _CHUNK), jnp.int32) for _ in range(_NBUF)]
            + [pltpu.VMEM((_SUB_E,), jnp.float32) for _ in range(_NBUF)]
            + [pltpu.VMEM((_SUB_E, _HALF), jnp.float32) for _ in range(_NBUF)]
            + [pltpu.VMEM((_ZCHUNK, _HALF), jnp.float32)]
            + [pltpu.SemaphoreType.DMA for _ in range(3 * _NBUF)]
        ),
        compiler_params=cp,
        name="lightgcn_sc",
    )
    _, outb = run(ego0, src1d, dst3d, val1d, bidx)

    out = jnp.concatenate([outb[0], outb[1]], axis=1)
    u = out[:_BATCH]
    pos = out[_BATCH:2 * _BATCH]
    neg = out[2 * _BATCH:]
    return (u, pos, neg)


# speedup vs baseline: 1.1191x; 1.0785x over previous
import jax
import jax.numpy as jnp
from jax import lax
from jax.experimental import pallas as pl
from jax.experimental.pallas import tpu as pltpu
from jax.experimental.pallas import tpu_sc as plsc

_N_USER = 60000
_N_ITEM = 40000
_N_NODES = _N_USER + _N_ITEM
_N_EDGES = 1600000
_HALF = 16
_N_LAYERS = 3
_BATCH = 4096

_CHUNK = 128
_SUB = 2
_SUB_E = _SUB * _CHUNK
_NS = 16
_NSB = _N_EDGES // _SUB_E
_NV = 392
_NB = 3 * _BATCH
_NB_CHUNKS = _NB // _CHUNK
_N_PAD = 100096
_ROWS_PER_SUB = _N_PAD // _NS
_ZCHUNK = 272
_NBUF = 4


def _sc_body(ego0_hbm, edge_hbm, val_hbm, users_hbm, pos_hbm, neg_hbm,
             egos_hbm, u_hbm, p_hbm, n_hbm, acc,
             s0, s1, s2, s3, d0, d1, d2, d3, v0, v1, v2, v3,
             r0, r1, r2, r3, zbuf,
             i0, i1, i2, i3, g0, g1, g2, g3, c0, c1, c2, c3):
    src_b = (s0, s1, s2, s3)
    dst_b = (d0, d1, d2, d3)
    val_b = (v0, v1, v2, v3)
    rows_b = (r0, r1, r2, r3)
    sem_i = (i0, i1, i2, i3)
    sem_g = (g0, g1, g2, g3)
    sem_s = (c0, c1, c2, c3)

    cid = lax.axis_index("c")
    sid = lax.axis_index("s")
    base_rows = sid * _ROWS_PER_SUB
    dummy = egos_hbm.at[0, 0, pl.ds(0, _SUB_E)]

    def blk(v):
        return sid + _NS * v

    def valid(v):
        return blk(v) < _NSB

    def issue_idx(v, S):
        b = blk(v)
        pltpu.async_copy(edge_hbm.at[0, pl.ds(b * _SUB_E, _SUB_E)],
                         src_b[S], sem_i[S])
        pltpu.async_copy(edge_hbm.at[1, pl.ds(b * _SUB_E, _CHUNK)],
                         dst_b[S].at[0], sem_i[S])
        pltpu.async_copy(edge_hbm.at[1, pl.ds(b * _SUB_E + _CHUNK, _CHUNK)],
                         dst_b[S].at[1], sem_i[S])
        pltpu.async_copy(val_hbm.at[pl.ds(b * _SUB_E, _SUB_E)],
                         val_b[S], sem_i[S])

    def drain_idx(S):
        pltpu.make_async_copy(
            edge_hbm.at[0, pl.ds(0, _SUB_E)], src_b[S], sem_i[S]).wait()
        pltpu.make_async_copy(
            edge_hbm.at[1, pl.ds(0, _CHUNK)], dst_b[S].at[0], sem_i[S]).wait()
        pltpu.make_async_copy(
            edge_hbm.at[1, pl.ds(0, _CHUNK)], dst_b[S].at[1], sem_i[S]).wait()
        pltpu.make_async_copy(
            val_hbm.at[pl.ds(0, _SUB_E)], val_b[S], sem_i[S]).wait()

    def interleave_idx(S):
        @pl.loop(0, _SUB_E, step=_HALF)
        def _(g):
            sl = pl.ds(g, _HALF)
            src_b[S].at[sl][...] = src_b[S].at[sl][...] * 2 + cid

    def issue_gathers(tbl, S):
        @pl.loop(0, _SUB)
        def _(k):
            pltpu.async_copy(
                tbl.at[src_b[S].at[pl.ds(k * _CHUNK, _CHUNK)]],
                rows_b[S].at[pl.ds(k * _CHUNK, _CHUNK)], sem_g[S])

    def drain_gathers(S):
        pltpu.make_async_copy(dummy, rows_b[S], sem_g[S]).wait()

    def multiply(S):
        rows, val = rows_b[S], val_b[S]

        @plsc.parallel_loop(0, _SUB_E, step=_HALF, unroll=2)
        def _(c):
            vv = val.at[pl.ds(c, _HALF)][...]
            for jj in range(_HALF):
                rows.at[c + jj][...] = rows.at[c + jj][...] * vv[jj]

    def issue_scatters(S):
        @pl.loop(0, _SUB)
        def _(k):
            pltpu.async_copy(
                rows_b[S].at[pl.ds(k * _CHUNK, _CHUNK)],
                acc.at[dst_b[S].at[k]], sem_s[S], add=True)

    def drain_scatters(S):
        pltpu.make_async_copy(dummy, rows_b[S], sem_s[S]).wait()

    def process(tbl, v, S, interleave):
        Sp = (S - 1) % _NBUF
        S2 = (S + 2) % _NBUF

        @pl.when(valid(v))
        def _():
            drain_idx(S)
            if interleave:
                interleave_idx(S)
            issue_gathers(tbl, S)

        @pl.when(jnp.logical_and(v >= 1, valid(v - 1)))
        def _():
            drain_gathers(Sp)
            multiply(Sp)

        @pl.when(jnp.logical_and(v >= 2, valid(v - 2)))
        def _():
            drain_scatters(S2)

        @pl.when(jnp.logical_and(v + 2 < _NV, valid(v + 2)))
        def _():
            issue_idx(v + 2, S2)

        @pl.when(jnp.logical_and(v >= 1, valid(v - 1)))
        def _():
            issue_scatters(Sp)

    def run_layer(tbl, out_view, interleave):
        issue_idx(0, 0)
        issue_idx(1, 1)

        @pl.loop(0, _NV, step=_NBUF)
        def _(v0_):
            for d in range(_NBUF):
                process(tbl, v0_ + d, d, interleave)

        Sl = (_NV - 1) % _NBUF

        @pl.when(valid(_NV - 1))
        def _():
            drain_gathers(Sl)
            multiply(Sl)

        @pl.when(valid(_NV - 2))
        def _():
            drain_scatters((_NV + 2) % _NBUF)

        @pl.when(valid(_NV - 1))
        def _():
            issue_scatters(Sl)
            drain_scatters(Sl)

        plsc.subcore_barrier()

        @pl.loop(0, _ROWS_PER_SUB // _ZCHUNK)
        def _(k):
            sl = pl.ds(base_rows + k * _ZCHUNK, _ZCHUNK)
            pltpu.sync_copy(acc.at[sl], out_view.at[sl])
            pltpu.sync_copy(zbuf, acc.at[sl])

        plsc.subcore_barrier()

    @pl.loop(0, _ZCHUNK)
    def _(i):
        zbuf.at[i][...] = jnp.zeros((_HALF,), jnp.float32)

    @pl.loop(0, _ROWS_PER_SUB // _ZCHUNK)
    def _(k):
        pltpu.sync_copy(zbuf, acc.at[pl.ds(base_rows + k * _ZCHUNK, _ZCHUNK)])

    plsc.subcore_barrier()

    run_layer(ego0_hbm, egos_hbm.at[0, cid], True)

    @pl.loop(1, _N_LAYERS)
    def _(t):
        run_layer(egos_hbm.at[t - 1, cid], egos_hbm.at[t, cid], False)

    nb_per_sub = _NB_CHUNKS // _NS

    @pl.loop(sid * nb_per_sub, (sid + 1) * nb_per_sub)
    def _(j):
        csl = pl.ds(0, _CHUNK)
        jr = j % 32
        bslice = pl.ds(jr * _CHUNK, _CHUNK)

        @pl.when(j < 32)
        def _():
            pltpu.sync_copy(users_hbm.at[bslice], s0.at[csl])

        @pl.when(jnp.logical_and(j >= 32, j < 64))
        def _():
            pltpu.sync_copy(pos_hbm.at[bslice], s0.at[csl])

        @pl.when(j >= 64)
        def _():
            pltpu.sync_copy(neg_hbm.at[bslice], s0.at[csl])

        ioff = jnp.where(j < 32, 0, _N_USER)

        @pl.loop(0, _CHUNK, step=_HALF)
        def _(g):
            sl = pl.ds(g, _HALF)
            x = s0.at[sl][...] + ioff
            s0.at[sl][...] = x
            s1.at[sl][...] = x * 2 + cid

        pltpu.async_copy(ego0_hbm.at[s1.at[csl]], r0.at[csl], g0).wait()

        for t in range(_N_LAYERS):
            pltpu.async_copy(egos_hbm.at[t, cid].at[s0.at[csl]],
                             r1.at[csl], g0).wait()

            @pl.loop(0, _CHUNK)
            def _(i):
                r0.at[i][...] = r0.at[i][...] + r1.at[i][...]

        @pl.loop(0, _CHUNK)
        def _(i):
            r0.at[i][...] = r0.at[i][...] * (1.0 / (_N_LAYERS + 1))

        osl = (pl.ds(jr * _CHUNK, _CHUNK), pl.ds(cid * _HALF, _HALF))

        @pl.when(j < 32)
        def _():
            pltpu.sync_copy(r0.at[csl], u_hbm.at[osl])

        @pl.when(jnp.logical_and(j >= 32, j < 64))
        def _():
            pltpu.sync_copy(r0.at[csl], p_hbm.at[osl])

        @pl.when(j >= 64)
        def _():
            pltpu.sync_copy(r0.at[csl], n_hbm.at[osl])


@jax.jit
def kernel(users, pos_items, neg_items, edge_index, edge_vals,
           user_emb, item_emb):
    ego0 = jnp.concatenate([user_emb, item_emb], axis=0)
    ego0 = ego0.reshape(2 * _N_NODES, _HALF)

    mesh = plsc.VectorSubcoreMesh(core_axis_name="c", subcore_axis_name="s")
    cp = pltpu.CompilerParams(
        needs_layout_passes=False, use_tc_tiling_on_sc=False)
    run = pl.kernel(
        _sc_body,
        out_type=(
            jax.ShapeDtypeStruct((_N_LAYERS, 2, _N_PAD, _HALF), jnp.float32),
            jax.ShapeDtypeStruct((_BATCH, 2 * _HALF), jnp.float32),
            jax.ShapeDtypeStruct((_BATCH, 2 * _HALF), jnp.float32),
            jax.ShapeDtypeStruct((_BATCH, 2 * _HALF), jnp.float32),
        ),
        mesh=mesh,
        scratch_types=(
            [pltpu.VMEM_SHARED((_N_PAD, _HALF), jnp.float32)]
            + [pltpu.VMEM((_SUB_E,), jnp.int32) for _ in range(_NBUF)]
            + [pltpu.VMEM((_SUB, _CHUNK), jnp.int32) for _ in range(_NBUF)]
            + [pltpu.VMEM((_SUB_E,), jnp.float32) for _ in range(_NBUF)]
            + [pltpu.VMEM((_SUB_E, _HALF), jnp.float32) for _ in range(_NBUF)]
            + [pltpu.VMEM((_ZCHUNK, _HALF), jnp.float32)]
            + [pltpu.SemaphoreType.DMA for _ in range(3 * _NBUF)]
        ),
        compiler_params=cp,
        name="lightgcn_sc",
    )
    _, u, pos, neg = run(ego0, edge_index, edge_vals,
                         users, pos_items, neg_items)
    return (u, pos, neg)


# speedup vs baseline: 1.1773x; 1.0520x over previous
import jax
import jax.numpy as jnp
from jax import lax
from jax.experimental import pallas as pl
from jax.experimental.pallas import tpu as pltpu
from jax.experimental.pallas import tpu_sc as plsc

_N_USER = 60000
_N_ITEM = 40000
_N_NODES = _N_USER + _N_ITEM
_N_EDGES = 1600000
_HALF = 16
_N_LAYERS = 3
_BATCH = 4096

_CHUNK = 128
_SUB = 2
_SUB_E = _SUB * _CHUNK
_NS = 16
_NSB = _N_EDGES // _SUB_E
_NV = 392
_NB = 3 * _BATCH
_NB_CHUNKS = _NB // _CHUNK
_N_PAD = 100096
_ROWS_PER_SUB = _N_PAD // _NS
_ZCHUNK = 272
_NBUF = 4


def _sc_body(ego0_hbm, edge_hbm, val_hbm, users_hbm, pos_hbm, neg_hbm,
             egos_hbm, u_hbm, p_hbm, n_hbm, acc,
             s0, s1, s2, s3, d0, d1, d2, d3, v0, v1, v2, v3,
             r0, r1, r2, r3, zbuf,
             i0, i1, i2, i3, g0, g1, g2, g3, c0, c1, c2, c3):
    src_b = (s0, s1, s2, s3)
    dst_b = (d0, d1, d2, d3)
    val_b = (v0, v1, v2, v3)
    rows_b = (r0, r1, r2, r3)
    sem_i = (i0, i1, i2, i3)
    sem_g = (g0, g1, g2, g3)
    sem_s = (c0, c1, c2, c3)

    cid = lax.axis_index("c")
    sid = lax.axis_index("s")
    base_rows = sid * _ROWS_PER_SUB
    dummy = egos_hbm.at[0, 0, pl.ds(0, _SUB_E)]

    def blk(v):
        return sid + _NS * v

    def valid(v):
        return blk(v) < _NSB

    def issue_idx(v, S):
        b = blk(v)
        pltpu.async_copy(edge_hbm.at[0, pl.ds(b * _SUB_E, _SUB_E)],
                         src_b[S], sem_i[S])
        pltpu.async_copy(edge_hbm.at[1, pl.ds(b * _SUB_E, _CHUNK)],
                         dst_b[S].at[0], sem_i[S])
        pltpu.async_copy(edge_hbm.at[1, pl.ds(b * _SUB_E + _CHUNK, _CHUNK)],
                         dst_b[S].at[1], sem_i[S])
        pltpu.async_copy(val_hbm.at[pl.ds(b * _SUB_E, _SUB_E)],
                         val_b[S], sem_i[S])

    def drain_idx(S):
        pltpu.make_async_copy(
            edge_hbm.at[0, pl.ds(0, _SUB_E)], src_b[S], sem_i[S]).wait()
        pltpu.make_async_copy(
            edge_hbm.at[1, pl.ds(0, _CHUNK)], dst_b[S].at[0], sem_i[S]).wait()
        pltpu.make_async_copy(
            edge_hbm.at[1, pl.ds(0, _CHUNK)], dst_b[S].at[1], sem_i[S]).wait()
        pltpu.make_async_copy(
            val_hbm.at[pl.ds(0, _SUB_E)], val_b[S], sem_i[S]).wait()

    def interleave_idx(S):
        @pl.loop(0, _SUB_E, step=_HALF)
        def _(g):
            sl = pl.ds(g, _HALF)
            src_b[S].at[sl][...] = src_b[S].at[sl][...] * 2 + cid

    def issue_gathers(tbl, S):
        @pl.loop(0, _SUB)
        def _(k):
            pltpu.async_copy(
                tbl.at[src_b[S].at[pl.ds(k * _CHUNK, _CHUNK)]],
                rows_b[S].at[pl.ds(k * _CHUNK, _CHUNK)], sem_g[S])

    def drain_gathers(S):
        pltpu.make_async_copy(dummy, rows_b[S], sem_g[S]).wait()

    def multiply(S):
        rows, val = rows_b[S], val_b[S]

        @plsc.parallel_loop(0, _SUB_E, step=_HALF, unroll=2)
        def _(c):
            vv = val.at[pl.ds(c, _HALF)][...]
            for jj in range(_HALF):
                rows.at[c + jj][...] = rows.at[c + jj][...] * vv[jj]

    def issue_scatters(S):
        @pl.loop(0, _SUB)
        def _(k):
            pltpu.async_copy(
                rows_b[S].at[pl.ds(k * _CHUNK, _CHUNK)],
                acc.at[dst_b[S].at[k]], sem_s[S], add=True)

    def drain_scatters(S):
        pltpu.make_async_copy(dummy, rows_b[S], sem_s[S]).wait()

    def process(tbl, v, S, interleave):
        Sp = (S - 1) % _NBUF
        S2 = (S + 2) % _NBUF

        @pl.when(valid(v))
        def _():
            drain_idx(S)
            if interleave:
                interleave_idx(S)
            issue_gathers(tbl, S)

        @pl.when(jnp.logical_and(v >= 1, valid(v - 1)))
        def _():
            drain_gathers(Sp)
            multiply(Sp)

        @pl.when(jnp.logical_and(v >= 2, valid(v - 2)))
        def _():
            drain_scatters(S2)

        @pl.when(jnp.logical_and(v + 2 < _NV, valid(v + 2)))
        def _():
            issue_idx(v + 2, S2)

        @pl.when(jnp.logical_and(v >= 1, valid(v - 1)))
        def _():
            issue_scatters(Sp)

    def run_layer(tbl, out_view, interleave):
        issue_idx(0, 0)
        issue_idx(1, 1)

        @pl.loop(0, _NV, step=_NBUF)
        def _(v0_):
            for d in range(_NBUF):
                process(tbl, v0_ + d, d, interleave)

        Sl = (_NV - 1) % _NBUF

        @pl.when(valid(_NV - 1))
        def _():
            drain_gathers(Sl)
            multiply(Sl)

        @pl.when(valid(_NV - 2))
        def _():
            drain_scatters((_NV + 2) % _NBUF)

        @pl.when(valid(_NV - 1))
        def _():
            issue_scatters(Sl)
            drain_scatters(Sl)

        plsc.subcore_barrier()

        @pl.loop(0, _ROWS_PER_SUB // _ZCHUNK)
        def _(k):
            sl = pl.ds(base_rows + k * _ZCHUNK, _ZCHUNK)
            pltpu.async_copy(acc.at[sl], out_view.at[sl], sem_i[0])

        @pl.loop(0, _ROWS_PER_SUB // _ZCHUNK)
        def _(k):
            sl = pl.ds(base_rows + k * _ZCHUNK, _ZCHUNK)
            pltpu.make_async_copy(acc.at[sl], out_view.at[sl],
                                  sem_i[0]).wait()

        @pl.loop(0, _ROWS_PER_SUB // _ZCHUNK)
        def _(k):
            sl = pl.ds(base_rows + k * _ZCHUNK, _ZCHUNK)
            pltpu.async_copy(zbuf, acc.at[sl], sem_i[1])

        @pl.loop(0, _ROWS_PER_SUB // _ZCHUNK)
        def _(k):
            sl = pl.ds(base_rows + k * _ZCHUNK, _ZCHUNK)
            pltpu.make_async_copy(zbuf, acc.at[sl], sem_i[1]).wait()

        plsc.subcore_barrier()

    @pl.loop(0, _ZCHUNK)
    def _(i):
        zbuf.at[i][...] = jnp.zeros((_HALF,), jnp.float32)

    @pl.loop(0, _ROWS_PER_SUB // _ZCHUNK)
    def _(k):
        pltpu.sync_copy(zbuf, acc.at[pl.ds(base_rows + k * _ZCHUNK, _ZCHUNK)])

    plsc.subcore_barrier()

    run_layer(ego0_hbm, egos_hbm.at[0, cid], True)

    @pl.loop(1, _N_LAYERS)
    def _(t):
        run_layer(egos_hbm.at[t - 1, cid], egos_hbm.at[t, cid], False)

    nb_per_sub = _NB_CHUNKS // _NS

    @pl.loop(sid * nb_per_sub, (sid + 1) * nb_per_sub)
    def _(j):
        csl = pl.ds(0, _CHUNK)
        jr = j % 32
        bslice = pl.ds(jr * _CHUNK, _CHUNK)

        @pl.when(j < 32)
        def _():
            pltpu.sync_copy(users_hbm.at[bslice], s0.at[csl])

        @pl.when(jnp.logical_and(j >= 32, j < 64))
        def _():
            pltpu.sync_copy(pos_hbm.at[bslice], s0.at[csl])

        @pl.when(j >= 64)
        def _():
            pltpu.sync_copy(neg_hbm.at[bslice], s0.at[csl])

        ioff = jnp.where(j < 32, 0, _N_USER)

        @pl.loop(0, _CHUNK, step=_HALF)
        def _(g):
            sl = pl.ds(g, _HALF)
            x = s0.at[sl][...] + ioff
            s0.at[sl][...] = x
            s1.at[sl][...] = x * 2 + cid

        pltpu.async_copy(ego0_hbm.at[s1.at[csl]], r0.at[csl], g0)
        for t in range(_N_LAYERS):
            pltpu.async_copy(egos_hbm.at[t, cid].at[s0.at[csl]],
                             rows_b[1 + t].at[csl], g0)
        for _t in range(_N_LAYERS + 1):
            pltpu.make_async_copy(dummy.at[pl.ds(0, _CHUNK)],
                                  r0.at[csl], g0).wait()

        @pl.loop(0, _CHUNK)
        def _(i):
            r0.at[i][...] = ((r0.at[i][...] + r1.at[i][...])
                             + (r2.at[i][...] + r3.at[i][...])
                             ) * (1.0 / (_N_LAYERS + 1))

        osl = (pl.ds(jr * _CHUNK, _CHUNK), pl.ds(cid * _HALF, _HALF))

        @pl.when(j < 32)
        def _():
            pltpu.sync_copy(r0.at[csl], u_hbm.at[osl])

        @pl.when(jnp.logical_and(j >= 32, j < 64))
        def _():
            pltpu.sync_copy(r0.at[csl], p_hbm.at[osl])

        @pl.when(j >= 64)
        def _():
            pltpu.sync_copy(r0.at[csl], n_hbm.at[osl])


@jax.jit
def kernel(users, pos_items, neg_items, edge_index, edge_vals,
           user_emb, item_emb):
    ego0 = jnp.concatenate([user_emb, item_emb], axis=0)
    ego0 = ego0.reshape(2 * _N_NODES, _HALF)

    mesh = plsc.VectorSubcoreMesh(core_axis_name="c", subcore_axis_name="s")
    cp = pltpu.CompilerParams(
        needs_layout_passes=False, use_tc_tiling_on_sc=False)
    run = pl.kernel(
        _sc_body,
        out_type=(
            jax.ShapeDtypeStruct((_N_LAYERS, 2, _N_PAD, _HALF), jnp.float32),
            jax.ShapeDtypeStruct((_BATCH, 2 * _HALF), jnp.float32),
            jax.ShapeDtypeStruct((_BATCH, 2 * _HALF), jnp.float32),
            jax.ShapeDtypeStruct((_BATCH, 2 * _HALF), jnp.float32),
        ),
        mesh=mesh,
        scratch_types=(
            [pltpu.VMEM_SHARED((_N_PAD, _HALF), jnp.float32)]
            + [pltpu.VMEM((_SUB_E,), jnp.int32) for _ in range(_NBUF)]
            + [pltpu.VMEM((_SUB, _CHUNK), jnp.int32) for _ in range(_NBUF)]
            + [pltpu.VMEM((_SUB_E,), jnp.float32) for _ in range(_NBUF)]
            + [pltpu.VMEM((_SUB_E, _HALF), jnp.float32) for _ in range(_NBUF)]
            + [pltpu.VMEM((_ZCHUNK, _HALF), jnp.float32)]
            + [pltpu.SemaphoreType.DMA for _ in range(3 * _NBUF)]
        ),
        compiler_params=cp,
        name="lightgcn_sc",
    )
    _, u, pos, neg = run(ego0, edge_index, edge_vals,
                         users, pos_items, neg_items)
    return (u, pos, neg)


# speedup vs baseline: 1.4401x; 1.2233x over previous
import jax
import jax.numpy as jnp
from jax import lax
from jax.experimental import pallas as pl
from jax.experimental.pallas import tpu as pltpu
from jax.experimental.pallas import tpu_sc as plsc

_N_USER = 60000
_N_ITEM = 40000
_N_NODES = _N_USER + _N_ITEM
_N_EDGES = 1600000
_HALF = 16
_N_LAYERS = 3
_BATCH = 4096

_CHUNK = 128
_SUB = 2
_SUB_E = _SUB * _CHUNK
_NS = 16
_NSB = _N_EDGES // _SUB_E
_NV = 395
_NB = 3 * _BATCH
_NB_CHUNKS = _NB // _CHUNK
_N_PAD = 100096
_ROWS_PER_SUB = _N_PAD // _NS
_ZCHUNK = 272
_NBUF = 5


def _sc_body(ego0_hbm, edge_hbm, val_hbm, users_hbm, pos_hbm, neg_hbm,
             egos_hbm, u_hbm, p_hbm, n_hbm, acc,
             s0, s1, s2, s3, s4, d0, d1, d2, d3, d4,
             v0, v1, v2, v3, v4, r0, r1, r2, r3, r4, zbuf,
             i0, i1, i2, i3, i4, g0, g1, g2, g3, g4,
             c0, c1, c2, c3, c4):
    src_b = (s0, s1, s2, s3, s4)
    dst_b = (d0, d1, d2, d3, d4)
    val_b = (v0, v1, v2, v3, v4)
    rows_b = (r0, r1, r2, r3, r4)
    sem_i = (i0, i1, i2, i3, i4)
    sem_g = (g0, g1, g2, g3, g4)
    sem_s = (c0, c1, c2, c3, c4)

    cid = lax.axis_index("c")
    sid = lax.axis_index("s")
    base_rows = sid * _ROWS_PER_SUB
    dummy = egos_hbm.at[0, 0, pl.ds(0, _SUB_E)]

    def blk(v):
        return sid + _NS * v

    def valid(v):
        return blk(v) < _NSB

    def issue_idx(v, S):
        b = blk(v)
        pltpu.async_copy(edge_hbm.at[0, pl.ds(b * _SUB_E, _SUB_E)],
                         src_b[S], sem_i[S])
        pltpu.async_copy(edge_hbm.at[1, pl.ds(b * _SUB_E, _CHUNK)],
                         dst_b[S].at[0], sem_i[S])
        pltpu.async_copy(edge_hbm.at[1, pl.ds(b * _SUB_E + _CHUNK, _CHUNK)],
                         dst_b[S].at[1], sem_i[S])
        pltpu.async_copy(val_hbm.at[pl.ds(b * _SUB_E, _SUB_E)],
                         val_b[S], sem_i[S])

    def drain_idx(S):
        pltpu.make_async_copy(
            edge_hbm.at[0, pl.ds(0, _SUB_E)], src_b[S], sem_i[S]).wait()
        pltpu.make_async_copy(
            edge_hbm.at[1, pl.ds(0, _CHUNK)], dst_b[S].at[0], sem_i[S]).wait()
        pltpu.make_async_copy(
            edge_hbm.at[1, pl.ds(0, _CHUNK)], dst_b[S].at[1], sem_i[S]).wait()
        pltpu.make_async_copy(
            val_hbm.at[pl.ds(0, _SUB_E)], val_b[S], sem_i[S]).wait()

    def interleave_idx(S):
        @pl.loop(0, _SUB_E, step=_HALF)
        def _(g):
            sl = pl.ds(g, _HALF)
            src_b[S].at[sl][...] = src_b[S].at[sl][...] * 2 + cid

    def issue_gathers(tbl, S):
        @pl.loop(0, _SUB)
        def _(k):
            pltpu.async_copy(
                tbl.at[src_b[S].at[pl.ds(k * _CHUNK, _CHUNK)]],
                rows_b[S].at[pl.ds(k * _CHUNK, _CHUNK)], sem_g[S])

    def drain_gathers(S):
        pltpu.make_async_copy(dummy, rows_b[S], sem_g[S]).wait()

    def multiply(S):
        rows, val = rows_b[S], val_b[S]

        @plsc.parallel_loop(0, _SUB_E, step=_HALF, unroll=2)
        def _(c):
            vv = val.at[pl.ds(c, _HALF)][...]
            for jj in range(_HALF):
                rows.at[c + jj][...] = rows.at[c + jj][...] * vv[jj]

    def issue_scatters(S):
        @pl.loop(0, _SUB)
        def _(k):
            pltpu.async_copy(
                rows_b[S].at[pl.ds(k * _CHUNK, _CHUNK)],
                acc.at[dst_b[S].at[k]], sem_s[S], add=True)

    def drain_scatters(S):
        pltpu.make_async_copy(dummy, rows_b[S], sem_s[S]).wait()

    def process(tbl, v, S, interleave):
        Sm2 = (S - 2) % _NBUF
        Sm3 = (S - 3) % _NBUF
        S2 = (S + 2) % _NBUF

        @pl.when(valid(v))
        def _():
            drain_idx(S)
            if interleave:
                interleave_idx(S)
            issue_gathers(tbl, S)

        @pl.when(jnp.logical_and(v >= 2, valid(v - 2)))
        def _():
            drain_gathers(Sm2)
            multiply(Sm2)

        @pl.when(jnp.logical_and(v >= 3, valid(v - 3)))
        def _():
            drain_scatters(Sm3)

        @pl.when(jnp.logical_and(v + 2 < _NV, valid(v + 2)))
        def _():
            issue_idx(v + 2, S2)

        @pl.when(jnp.logical_and(v >= 2, valid(v - 2)))
        def _():
            issue_scatters(Sm2)

    def run_layer(tbl, out_view, interleave):
        issue_idx(0, 0)
        issue_idx(1, 1)

        @pl.loop(0, _NV, step=_NBUF)
        def _(v0_):
            for d in range(_NBUF):
                process(tbl, v0_ + d, d, interleave)

        plsc.subcore_barrier()

        @pl.loop(0, _ROWS_PER_SUB // _ZCHUNK)
        def _(k):
            sl = pl.ds(base_rows + k * _ZCHUNK, _ZCHUNK)
            pltpu.async_copy(acc.at[sl], out_view.at[sl], sem_i[0])

        @pl.loop(0, _ROWS_PER_SUB // _ZCHUNK)
        def _(k):
            sl = pl.ds(base_rows + k * _ZCHUNK, _ZCHUNK)
            pltpu.make_async_copy(acc.at[sl], out_view.at[sl],
                                  sem_i[0]).wait()

        @pl.loop(0, _ROWS_PER_SUB // _ZCHUNK)
        def _(k):
            sl = pl.ds(base_rows + k * _ZCHUNK, _ZCHUNK)
            pltpu.async_copy(zbuf, acc.at[sl], sem_i[1])

        @pl.loop(0, _ROWS_PER_SUB // _ZCHUNK)
        def _(k):
            sl = pl.ds(base_rows + k * _ZCHUNK, _ZCHUNK)
            pltpu.make_async_copy(zbuf, acc.at[sl], sem_i[1]).wait()

        plsc.subcore_barrier()

    @pl.loop(0, _ZCHUNK)
    def _(i):
        zbuf.at[i][...] = jnp.zeros((_HALF,), jnp.float32)

    @pl.loop(0, _ROWS_PER_SUB // _ZCHUNK)
    def _(k):
        pltpu.sync_copy(zbuf, acc.at[pl.ds(base_rows + k * _ZCHUNK, _ZCHUNK)])

    plsc.subcore_barrier()

    run_layer(ego0_hbm, egos_hbm.at[0, cid], True)

    @pl.loop(1, _N_LAYERS)
    def _(t):
        run_layer(egos_hbm.at[t - 1, cid], egos_hbm.at[t, cid], False)

    nb_per_sub = _NB_CHUNKS // _NS

    @pl.loop(sid * nb_per_sub, (sid + 1) * nb_per_sub)
    def _(j):
        csl = pl.ds(0, _CHUNK)
        jr = j % 32
        bslice = pl.ds(jr * _CHUNK, _CHUNK)

        @pl.when(j < 32)
        def _():
            pltpu.sync_copy(users_hbm.at[bslice], s0.at[csl])

        @pl.when(jnp.logical_and(j >= 32, j < 64))
        def _():
            pltpu.sync_copy(pos_hbm.at[bslice], s0.at[csl])

        @pl.when(j >= 64)
        def _():
            pltpu.sync_copy(neg_hbm.at[bslice], s0.at[csl])

        ioff = jnp.where(j < 32, 0, _N_USER)

        @pl.loop(0, _CHUNK, step=_HALF)
        def _(g):
            sl = pl.ds(g, _HALF)
            x = s0.at[sl][...] + ioff
            s0.at[sl][...] = x
            s1.at[sl][...] = x * 2 + cid

        pltpu.async_copy(ego0_hbm.at[s1.at[csl]], r0.at[csl], g0)
        for t in range(_N_LAYERS):
            pltpu.async_copy(egos_hbm.at[t, cid].at[s0.at[csl]],
                             rows_b[1 + t].at[csl], g0)
        for _t in range(_N_LAYERS + 1):
            pltpu.make_async_copy(dummy.at[pl.ds(0, _CHUNK)],
                                  r0.at[csl], g0).wait()

        @pl.loop(0, _CHUNK)
        def _(i):
            r0.at[i][...] = ((r0.at[i][...] + r1.at[i][...])
                             + (r2.at[i][...] + r3.at[i][...])
                             ) * (1.0 / (_N_LAYERS + 1))

        osl = (pl.ds(jr * _CHUNK, _CHUNK), pl.ds(cid * _HALF, _HALF))

        @pl.when(j < 32)
        def _():
            pltpu.sync_copy(r0.at[csl], u_hbm.at[osl])

        @pl.when(jnp.logical_and(j >= 32, j < 64))
        def _():
            pltpu.sync_copy(r0.at[csl], p_hbm.at[osl])

        @pl.when(j >= 64)
        def _():
            pltpu.sync_copy(r0.at[csl], n_hbm.at[osl])


@jax.jit
def kernel(users, pos_items, neg_items, edge_index, edge_vals,
           user_emb, item_emb):
    ego0 = jnp.concatenate([user_emb, item_emb], axis=0)
    ego0 = ego0.reshape(2 * _N_NODES, _HALF)

    mesh = plsc.VectorSubcoreMesh(core_axis_name="c", subcore_axis_name="s")
    cp = pltpu.CompilerParams(
        needs_layout_passes=False, use_tc_tiling_on_sc=False)
    run = pl.kernel(
        _sc_body,
        out_type=(
            jax.ShapeDtypeStruct((_N_LAYERS, 2, _N_PAD, _HALF), jnp.float32),
            jax.ShapeDtypeStruct((_BATCH, 2 * _HALF), jnp.float32),
            jax.ShapeDtypeStruct((_BATCH, 2 * _HALF), jnp.float32),
            jax.ShapeDtypeStruct((_BATCH, 2 * _HALF), jnp.float32),
        ),
        mesh=mesh,
        scratch_types=(
            [pltpu.VMEM_SHARED((_N_PAD, _HALF), jnp.float32)]
            + [pltpu.VMEM((_SUB_E,), jnp.int32) for _ in range(_NBUF)]
            + [pltpu.VMEM((_SUB, _CHUNK), jnp.int32) for _ in range(_NBUF)]
            + [pltpu.VMEM((_SUB_E,), jnp.float32) for _ in range(_NBUF)]
            + [pltpu.VMEM((_SUB_E, _HALF), jnp.float32) for _ in range(_NBUF)]
            + [pltpu.VMEM((_ZCHUNK, _HALF), jnp.float32)]
            + [pltpu.SemaphoreType.DMA for _ in range(3 * _NBUF)]
        ),
        compiler_params=cp,
        name="lightgcn_sc",
    )
    _, u, pos, neg = run(ego0, edge_index, edge_vals,
                         users, pos_items, neg_items)
    return (u, pos, neg)
